# baseline (speedup 1.0000x reference)
import functools

import jax
import jax.numpy as jnp
from jax import lax
from jax.experimental import pallas as pl
from jax.experimental.pallas import tpu as pltpu
from jax.experimental.pallas import tpu_sc as plsc

N_USERS = 25000
N_NODES = 50000
D = 64
HD = 32
N_EDGES = 800000

NC = 2
NS = 16
CHUNK = 128
BLK = 8
EPT = 51200
NBLK = EPT // (CHUNK * BLK)
E_PAD = EPT * NS
ACC_ROWS = 50000
ZPT = ACC_ROWS // NS
NBUF = 6
DEPTH = 5

_mesh = plsc.VectorSubcoreMesh(core_axis_name="c", subcore_axis_name="s")


def _spmm_body(emb_h, idx_h, val_h, out_h, idx_b, val_b, bufs, acc,
               gsems, ssems, isem, wsem):
    cid = lax.axis_index("c")
    sid = lax.axis_index("s")
    zero16 = jnp.zeros((16,), jnp.float32)

    @pl.loop(0, CHUNK)
    def _(r):
        for q in range(2):
            bufs[0, r, q * 16:(q + 1) * 16] = zero16

    zb = sid * ZPT
    zd = []
    for i in range(24):
        zd.append(pltpu.async_copy(
            bufs.at[0], acc.at[pl.ds(zb + i * 128, 128)], wsem))
        if i >= 8:
            zd[i - 8].wait()
    zd.append(pltpu.async_copy(
        bufs.at[0, pl.ds(0, ZPT - 24 * 128)],
        acc.at[pl.ds(zb + 24 * 128, ZPT - 24 * 128)], wsem))
    for i in range(16, 25):
        zd[i].wait()

    pltpu.async_copy(idx_h.at[cid, sid, 0], idx_b.at[0], isem)
    pltpu.async_copy(val_h.at[sid, 0], val_b.at[0], isem)

    plsc.subcore_barrier()

    @pl.loop(0, NBLK, step=2)
    def _(g):
        for s in range(2):
            b = g + s
            pltpu.make_async_copy(idx_h.at[0, 0, 0], idx_b.at[0], isem).wait()
            pltpu.make_async_copy(val_h.at[0, 0], val_b.at[0], isem).wait()
            bn = jnp.minimum(b + 1, NBLK - 1)
            pltpu.async_copy(idx_h.at[cid, sid, bn], idx_b.at[1 - s], isem)
            pltpu.async_copy(val_h.at[sid, bn], val_b.at[1 - s], isem)

            def scale(buf, j, s=s):
                @pl.loop(0, CHUNK // 16)
                def _(gq):
                    vals = val_b[s, j, pl.ds(gq * 16, 16)]
                    for ln in range(16):
                        v = vals[ln]
                        e = gq * 16 + ln
                        for q in range(2):
                            sl = buf[e, q * 16:(q + 1) * 16]
                            buf[e, q * 16:(q + 1) * 16] = sl * v

            gd = [None] * BLK
            sd = [None] * BLK
            for j in range(DEPTH):
                gd[j] = pltpu.async_copy(
                    emb_h.at[idx_b.at[s, 0, j]], bufs.at[j % NBUF],
                    gsems[j % NBUF])
            for j in range(BLK):
                nb = j % NBUF
                gd[j].wait()
                scale(bufs.at[nb], j)
                sd[j] = pltpu.async_copy(
                    bufs.at[nb], acc.at[idx_b.at[s, 1, j]], ssems[nb],
                    add=True)
                jn = j + DEPTH
                if jn < BLK:
                    if j >= 1:
                        sd[j - 1].wait()
                    gd[jn] = pltpu.async_copy(
                        emb_h.at[idx_b.at[s, 0, jn]], bufs.at[jn % NBUF],
                        gsems[jn % NBUF])
            for j in range(BLK - DEPTH - 1, BLK):
                sd[j].wait()

    pltpu.make_async_copy(idx_h.at[0, 0, 0], idx_b.at[0], isem).wait()
    pltpu.make_async_copy(val_h.at[0, 0], val_b.at[0], isem).wait()

    plsc.subcore_barrier()

    wd = []
    for i in range(25):
        r = (sid * 25 + i) * 125
        wd.append(pltpu.async_copy(
            acc.at[pl.ds(r, 125)],
            out_h.at[pl.ds(cid * N_NODES + r, 125)], wsem))
        if i >= 8:
            wd[i - 8].wait()
    for i in range(17, 25):
        wd[i].wait()


_spmm = pl.kernel(
    _spmm_body,
    out_type=jax.ShapeDtypeStruct((NC * N_NODES, HD), jnp.float32),
    mesh=_mesh,
    compiler_params=pltpu.CompilerParams(use_tc_tiling_on_sc=False),
    scratch_types=[
        pltpu.VMEM((2, 2, BLK, CHUNK), jnp.int32),
        pltpu.VMEM((2, BLK, CHUNK), jnp.float32),
        pltpu.VMEM((NBUF, CHUNK, HD), jnp.float32),
        pltpu.VMEM_SHARED((ACC_ROWS, HD), jnp.float32),
        [pltpu.SemaphoreType.DMA] * NBUF,
        [pltpu.SemaphoreType.DMA] * NBUF,
        pltpu.SemaphoreType.DMA,
        pltpu.SemaphoreType.DMA,
    ],
)


def _combine_body(e, a0, b0, c0, a1, b1, c1, o):
    o[:, :HD] = (e[:, :HD] + a0[...] + b0[...] + c0[...]) * 0.25
    o[:, HD:] = (e[:, HD:] + a1[...] + b1[...] + c1[...]) * 0.25


_RB = 2000

_combine = pl.pallas_call(
    _combine_body,
    out_shape=jax.ShapeDtypeStruct((N_NODES, D), jnp.float32),
    grid=(25,),
    in_specs=[pl.BlockSpec((_RB, D), lambda i: (i, 0))]
    + [pl.BlockSpec((_RB, HD), lambda i: (i, 0))] * 3
    + [pl.BlockSpec((_RB, HD), lambda i: (i + 25, 0))] * 3,
    out_specs=pl.BlockSpec((_RB, D), lambda i: (i, 0)),
)


def kernel(emb, adj_idx, adj_val):
    row = adj_idx[0].astype(jnp.int32)
    col = adj_idx[1].astype(jnp.int32)
    val = adj_val.astype(jnp.float32)

    npad = E_PAD - N_EDGES
    ar = jnp.arange(npad, dtype=jnp.int32)
    col_p = jnp.concatenate([col, ar % N_NODES])
    row_p = jnp.concatenate([row, (ar * 7) % N_NODES])
    val_p = jnp.concatenate([val, jnp.zeros((npad,), jnp.float32)])

    shape = (NS, NBLK, BLK, CHUNK)
    dst_r = row_p.reshape(shape)

    def pack_for(colmap):
        per_c = [jnp.stack([colmap(c).reshape(shape), dst_r], axis=2)
                 for c in range(NC)]
        return jnp.stack(per_c)

    pack1 = pack_for(lambda c: 2 * col_p + c)
    pack2 = pack_for(lambda c: col_p + c * N_NODES)
    val_h = val_p.reshape(shape)

    e0r = emb.reshape(NC * N_NODES, HD)
    p1 = _spmm(e0r, pack1, val_h)
    p2 = _spmm(p1, pack2, val_h)
    p3 = _spmm(p2, pack2, val_h)
    out = _combine(emb, p1, p2, p3, p1, p2, p3)
    return (out[:N_USERS], out[N_USERS:])


# speedup vs baseline: 1.1014x; 1.1014x over previous
import functools

import jax
import jax.numpy as jnp
from jax import lax
from jax.experimental import pallas as pl
from jax.experimental.pallas import tpu as pltpu
from jax.experimental.pallas import tpu_sc as plsc

N_USERS = 25000
N_NODES = 50000
D = 64
HD = 32
N_EDGES = 800000

NC = 2
NS = 16
CHUNK = 128
BLK = 10
EPT = 51200
NBLK = EPT // (CHUNK * BLK)
E_PAD = EPT * NS
ACC_ROWS = 50000
ZPT = ACC_ROWS // NS
NBUF = 5
DEPTH = 4

_mesh = plsc.VectorSubcoreMesh(core_axis_name="c", subcore_axis_name="s")


def _spmm_body(emb_h, idx_h, val_h, out_h, idx_b, val_b, bufs, acc,
               gsems, ssems, isem, wsem):
    cid = lax.axis_index("c")
    sid = lax.axis_index("s")
    zero16 = jnp.zeros((16,), jnp.float32)

    @pl.loop(0, CHUNK)
    def _(r):
        for q in range(2):
            bufs[0, r, q * 16:(q + 1) * 16] = zero16

    zb = sid * ZPT
    zd = []
    for i in range(24):
        zd.append(pltpu.async_copy(
            bufs.at[0], acc.at[pl.ds(zb + i * 128, 128)], wsem))
        if i >= 8:
            zd[i - 8].wait()
    zd.append(pltpu.async_copy(
        bufs.at[0, pl.ds(0, ZPT - 24 * 128)],
        acc.at[pl.ds(zb + 24 * 128, ZPT - 24 * 128)], wsem))
    for i in range(16, 25):
        zd[i].wait()

    def gath(c5, slot, jj):
        pltpu.async_copy(
            emb_h.at[idx_b.at[slot, 0, jj]], bufs.at[c5], gsems[c5])

    def waitg(c5):
        pltpu.make_async_copy(
            emb_h.at[pl.ds(0, CHUNK)], bufs.at[c5], gsems[c5]).wait()

    def scat(c5, slot, j):
        pltpu.async_copy(
            bufs.at[c5], acc.at[idx_b.at[slot, 1, j]], ssems[c5], add=True)

    def waits(c5):
        pltpu.make_async_copy(
            emb_h.at[pl.ds(0, CHUNK)], bufs.at[c5], ssems[c5]).wait()

    def stage(bn, slot):
        pltpu.async_copy(idx_h.at[cid, sid, bn], idx_b.at[slot], isem)
        pltpu.async_copy(val_h.at[sid, bn], val_b.at[slot], isem)

    def sdrain():
        pltpu.make_async_copy(idx_h.at[0, 0, 0], idx_b.at[0], isem).wait()
        pltpu.make_async_copy(val_h.at[0, 0], val_b.at[0], isem).wait()

    def scale(c5, slot, j):
        buf = bufs.at[c5]

        @pl.loop(0, CHUNK // 16)
        def _(gq):
            vals = val_b[slot, j, pl.ds(gq * 16, 16)]
            for ln in range(16):
                v = vals[ln]
                e = gq * 16 + ln
                for q in range(2):
                    sl = buf[e, q * 16:(q + 1) * 16]
                    buf[e, q * 16:(q + 1) * 16] = sl * v

    def block_body(b, slot, first=False, last=False):
        for j in range(BLK):
            c5 = j % NBUF
            if j == 1 and not last:
                stage(b + 1, 1 - slot)
            waitg(c5)
            scale(c5, slot, j)
            scat(c5, slot, j)
            if not (first and j == 0):
                waits((j - 1) % NBUF)
            if j == BLK - DEPTH and not last:
                sdrain()
            jj = j + DEPTH
            if not last or jj < BLK:
                if jj < BLK:
                    gath(jj % NBUF, slot, jj)
                else:
                    gath((jj - BLK) % NBUF, 1 - slot, jj - BLK)

    stage(0, 0)

    plsc.subcore_barrier()

    sdrain()
    for j in range(DEPTH):
        gath(j % NBUF, 0, j)

    block_body(0, 0, first=True)

    @pl.loop(1, NBLK - 1, step=2)
    def _(g):
        block_body(g, 1)
        block_body(g + 1, 0)

    block_body(NBLK - 1, (NBLK - 1) % 2, last=True)
    waits((BLK - 1) % NBUF)

    plsc.subcore_barrier()

    wd = []
    for i in range(25):
        r = (sid * 25 + i) * 125
        wd.append(pltpu.async_copy(
            acc.at[pl.ds(r, 125)],
            out_h.at[pl.ds(cid * N_NODES + r, 125)], wsem))
        if i >= 8:
            wd[i - 8].wait()
    for i in range(17, 25):
        wd[i].wait()


_spmm = pl.kernel(
    _spmm_body,
    out_type=jax.ShapeDtypeStruct((NC * N_NODES, HD), jnp.float32),
    mesh=_mesh,
    compiler_params=pltpu.CompilerParams(use_tc_tiling_on_sc=False),
    scratch_types=[
        pltpu.VMEM((2, 2, BLK, CHUNK), jnp.int32),
        pltpu.VMEM((2, BLK, CHUNK), jnp.float32),
        pltpu.VMEM((NBUF, CHUNK, HD), jnp.float32),
        pltpu.VMEM_SHARED((ACC_ROWS, HD), jnp.float32),
        [pltpu.SemaphoreType.DMA] * NBUF,
        [pltpu.SemaphoreType.DMA] * NBUF,
        pltpu.SemaphoreType.DMA,
        pltpu.SemaphoreType.DMA,
    ],
)


def _combine_body(e, a0, b0, c0, a1, b1, c1, o):
    o[:, :HD] = (e[:, :HD] + a0[...] + b0[...] + c0[...]) * 0.25
    o[:, HD:] = (e[:, HD:] + a1[...] + b1[...] + c1[...]) * 0.25


_RB = 2000

_combine = pl.pallas_call(
    _combine_body,
    out_shape=jax.ShapeDtypeStruct((N_NODES, D), jnp.float32),
    grid=(25,),
    in_specs=[pl.BlockSpec((_RB, D), lambda i: (i, 0))]
    + [pl.BlockSpec((_RB, HD), lambda i: (i, 0))] * 3
    + [pl.BlockSpec((_RB, HD), lambda i: (i + 25, 0))] * 3,
    out_specs=pl.BlockSpec((_RB, D), lambda i: (i, 0)),
)


def kernel(emb, adj_idx, adj_val):
    row = adj_idx[0].astype(jnp.int32)
    col = adj_idx[1].astype(jnp.int32)
    val = adj_val.astype(jnp.float32)

    npad = E_PAD - N_EDGES
    ar = jnp.arange(npad, dtype=jnp.int32)
    col_p = jnp.concatenate([col, ar % N_NODES])
    row_p = jnp.concatenate([row, (ar * 7) % N_NODES])
    val_p = jnp.concatenate([val, jnp.zeros((npad,), jnp.float32)])

    shape = (NS, NBLK, BLK, CHUNK)
    dst_r = row_p.reshape(shape)

    def pack_for(colmap):
        per_c = [jnp.stack([colmap(c).reshape(shape), dst_r], axis=2)
                 for c in range(NC)]
        return jnp.stack(per_c)

    pack1 = pack_for(lambda c: 2 * col_p + c)
    pack2 = pack_for(lambda c: col_p + c * N_NODES)
    val_h = val_p.reshape(shape)

    e0r = emb.reshape(NC * N_NODES, HD)
    p1 = _spmm(e0r, pack1, val_h)
    p2 = _spmm(p1, pack2, val_h)
    p3 = _spmm(p2, pack2, val_h)
    out = _combine(emb, p1, p2, p3, p1, p2, p3)
    return (out[:N_USERS], out[N_USERS:])
